# baseline (speedup 1.0000x reference)
import functools

import jax
import jax.numpy as jnp
from jax.experimental import pallas as pl
from jax.experimental.pallas import tpu as pltpu

_N = 8192
_D = 32
_TOKENS = 8192
_T = 512
_NT = _TOKENS // _T
_CHUNK = 2048
_NCHUNK = _N // _CHUNK
_COMMIT = 0.25


_HI = 128
_LO = 64


def _body(zf_ref, sz_ref, se_ref, nm_ref, emb_ref, embt_ref,
          quant_ref, loss_ref, perp_ref,
          counts_ref, lacc_ref, wacc_ref):
    i = pl.program_id(0)

    @pl.when(i == 0)
    def _init():
        counts_ref[...] = jnp.zeros_like(counts_ref)
        lacc_ref[...] = jnp.zeros_like(lacc_ref)
        wacc_ref[...] = jnp.zeros_like(wacc_ref)

    zfb = zf_ref[...]
    zbb = zfb.astype(jnp.bfloat16)
    ebb = emb_ref[...].astype(jnp.bfloat16)
    szb = sz_ref[...]

    acc_v = None
    acc_i = None
    for c in range(_NCHUNK):
        lo = c * _CHUNK
        mm = jax.lax.dot_general(
            zbb, ebb[lo:lo + _CHUNK, :],
            (((1,), (1,)), ((), ())),
            preferred_element_type=jnp.float32)
        dch = (szb + se_ref[:, lo:lo + _CHUNK]) - 2.0 * mm
        mc = jnp.min(dch, axis=1, keepdims=True)
        io = jax.lax.broadcasted_iota(jnp.int32, (_T, _CHUNK), 1)
        ic = jnp.min(jnp.where(dch == mc, io, _N), axis=1, keepdims=True) + lo
        if c == 0:
            acc_v = mc.astype(jnp.bfloat16).astype(jnp.float32)
            acc_i = ic
        else:
            keep = acc_v <= mc
            acc_i = jnp.where(keep, acc_i, ic)
            acc_v = jnp.where(keep, acc_v, mc)
            acc_v = acc_v.astype(jnp.bfloat16).astype(jnp.float32)

    hi = jax.lax.shift_right_logical(acc_i, 6)
    lo_ = jax.lax.bitwise_and(acc_i, _LO - 1)
    oh_hi = (jax.lax.broadcasted_iota(jnp.int32, (_T, _HI), 1)
             == hi).astype(jnp.float32)
    oh_lo = (jax.lax.broadcasted_iota(jnp.int32, (_T, _LO), 1)
             == lo_).astype(jnp.bfloat16)

    g1 = jax.lax.dot_general(
        oh_lo, embt_ref[...].astype(jnp.bfloat16),
        (((1,), (0,)), ((), ())),
        preferred_element_type=jnp.float32)
    zq = jnp.sum(g1.reshape(_T, _HI, _D) * oh_hi[:, :, None], axis=1)
    quant_ref[...] = zfb + (zq - zfb)

    lacc_ref[...] += jnp.sum((zq - zfb) ** 2).reshape(1, 1)
    w = nm_ref[...]
    wacc_ref[...] += jnp.sum(w).reshape(1, 1)
    counts_ref[...] += jax.lax.dot_general(
        (oh_hi * w).astype(jnp.bfloat16), oh_lo,
        (((0,), (0,)), ((), ())),
        preferred_element_type=jnp.float32)

    @pl.when(i == _NT - 1)
    def _finish():
        denom = jnp.maximum(wacc_ref[0, 0], 1.0)
        avg = counts_ref[...] / denom
        ent = jnp.sum(avg * jnp.log(avg + 1e-10))
        perp_ref[...] = jnp.exp(-ent).reshape(1, 1)
        loss_ref[...] = (_COMMIT * (lacc_ref[0, 0]
                                    / jnp.float32(_TOKENS * _D))).reshape(1, 1)


@functools.partial(jax.jit, static_argnames=())
def kernel(z, track_pad_mask, emb):
    input_shape = z.shape
    zf = z.reshape(-1, z.shape[-1])
    mask = track_pad_mask.reshape(-1)
    sz = jnp.sum(zf ** 2, axis=1, keepdims=True)
    se = jnp.sum(emb ** 2, axis=1).reshape(1, -1)
    notmask = jnp.logical_not(mask).astype(zf.dtype).reshape(-1, 1)
    embt = jnp.transpose(emb.reshape(_HI, _LO, _D), (1, 0, 2)).reshape(_LO, _HI * _D)

    quant, loss, perp = pl.pallas_call(
        _body,
        grid=(_NT,),
        in_specs=[
            pl.BlockSpec((_T, _D), lambda i: (i, 0)),
            pl.BlockSpec((_T, 1), lambda i: (i, 0)),
            pl.BlockSpec((1, _N), lambda i: (0, 0)),
            pl.BlockSpec((_T, 1), lambda i: (i, 0)),
            pl.BlockSpec((_N, _D), lambda i: (0, 0)),
            pl.BlockSpec((_LO, _HI * _D), lambda i: (0, 0)),
        ],
        out_specs=[
            pl.BlockSpec((_T, _D), lambda i: (i, 0)),
            pl.BlockSpec((1, 1), lambda i: (0, 0)),
            pl.BlockSpec((1, 1), lambda i: (0, 0)),
        ],
        out_shape=[
            jax.ShapeDtypeStruct((_TOKENS, _D), jnp.float32),
            jax.ShapeDtypeStruct((1, 1), jnp.float32),
            jax.ShapeDtypeStruct((1, 1), jnp.float32),
        ],
        scratch_shapes=[
            pltpu.VMEM((_HI, _LO), jnp.float32),
            pltpu.VMEM((1, 1), jnp.float32),
            pltpu.VMEM((1, 1), jnp.float32),
        ],
        compiler_params=pltpu.CompilerParams(
            dimension_semantics=("arbitrary",)),
    )(zf, sz, se, notmask, emb, embt)

    return quant.reshape(input_shape), loss.reshape(()), perp.reshape(())


# speedup vs baseline: 2.6395x; 2.6395x over previous
import functools

import jax
import jax.numpy as jnp
from jax.experimental import pallas as pl
from jax.experimental.pallas import tpu as pltpu

_N = 8192
_D = 32
_TOKENS = 8192
_T = 512
_NT = _TOKENS // _T
_CHUNK = 2048
_NCHUNK = _N // _CHUNK
_COMMIT = 0.25


_HI = 128
_LO = 64


def _body(zf_ref, sz_ref, se_ref, nm_ref, emb_ref, embt_ref,
          quant_ref, loss_ref, perp_ref,
          counts_ref, lacc_ref, wacc_ref):
    i = pl.program_id(0)

    @pl.when(i == 0)
    def _init():
        counts_ref[...] = jnp.zeros_like(counts_ref)
        lacc_ref[...] = jnp.zeros_like(lacc_ref)
        wacc_ref[...] = jnp.zeros_like(wacc_ref)

    zfb = zf_ref[...]
    zbb = zfb.astype(jnp.bfloat16)
    ebb = emb_ref[...].astype(jnp.bfloat16)
    szb = sz_ref[...]

    acc_v = None
    acc_i = None
    for c in range(_NCHUNK):
        lo = c * _CHUNK
        mm = jax.lax.dot_general(
            zbb, ebb[lo:lo + _CHUNK, :],
            (((1,), (1,)), ((), ())),
            preferred_element_type=jnp.float32)
        dch = (szb + se_ref[:, lo:lo + _CHUNK]) - 2.0 * mm
        mc = jnp.min(dch, axis=1, keepdims=True)
        io = jax.lax.broadcasted_iota(jnp.int32, (_T, _CHUNK), 1)
        ic = jnp.min(jnp.where(dch == mc, io, _N), axis=1, keepdims=True) + lo
        if c == 0:
            acc_v = mc.astype(jnp.bfloat16).astype(jnp.float32)
            acc_i = ic
        else:
            keep = acc_v <= mc
            acc_i = jnp.where(keep, acc_i, ic)
            acc_v = jnp.where(keep, acc_v, mc)
            acc_v = acc_v.astype(jnp.bfloat16).astype(jnp.float32)

    hi = jax.lax.shift_right_logical(acc_i, 6)
    lo_ = jax.lax.bitwise_and(acc_i, _LO - 1)
    oh_hi = (jax.lax.broadcasted_iota(jnp.int32, (_T, _HI), 1)
             == hi).astype(jnp.float32)
    oh_lo = (jax.lax.broadcasted_iota(jnp.int32, (_T, _LO), 1)
             == lo_).astype(jnp.bfloat16)

    h = jax.lax.dot_general(
        oh_hi.astype(jnp.bfloat16), embt_ref[...].astype(jnp.bfloat16),
        (((1,), (0,)), ((), ())),
        preferred_element_type=jnp.float32)
    lane = jax.lax.broadcasted_iota(jnp.int32, (_T, _LO * _D), 1)
    masked = jnp.where(jax.lax.shift_right_logical(lane, 5) == lo_,
                       h, 0.0).astype(jnp.bfloat16)
    sel = (jax.lax.bitwise_and(
        jax.lax.broadcasted_iota(jnp.int32, (_LO * _D, _D), 0), _D - 1)
        == jax.lax.broadcasted_iota(jnp.int32, (_LO * _D, _D), 1)
    ).astype(jnp.bfloat16)
    zq = jax.lax.dot_general(
        masked, sel, (((1,), (0,)), ((), ())),
        preferred_element_type=jnp.float32)
    quant_ref[...] = zfb + (zq - zfb)

    lacc_ref[...] += jnp.sum((zq - zfb) ** 2).reshape(1, 1)
    w = nm_ref[...]
    wacc_ref[...] += jnp.sum(w).reshape(1, 1)
    counts_ref[...] += jax.lax.dot_general(
        (oh_hi * w).astype(jnp.bfloat16), oh_lo,
        (((0,), (0,)), ((), ())),
        preferred_element_type=jnp.float32)

    @pl.when(i == _NT - 1)
    def _finish():
        denom = jnp.maximum(wacc_ref[0, 0], 1.0)
        avg = counts_ref[...] / denom
        ent = jnp.sum(avg * jnp.log(avg + 1e-10))
        perp_ref[...] = jnp.exp(-ent).reshape(1, 1)
        loss_ref[...] = (_COMMIT * (lacc_ref[0, 0]
                                    / jnp.float32(_TOKENS * _D))).reshape(1, 1)


@functools.partial(jax.jit, static_argnames=())
def kernel(z, track_pad_mask, emb):
    input_shape = z.shape
    zf = z.reshape(-1, z.shape[-1])
    mask = track_pad_mask.reshape(-1)
    sz = jnp.sum(zf ** 2, axis=1, keepdims=True)
    se = jnp.sum(emb ** 2, axis=1).reshape(1, -1)
    notmask = jnp.logical_not(mask).astype(zf.dtype).reshape(-1, 1)
    embt = emb.reshape(_HI, _LO * _D)

    quant, loss, perp = pl.pallas_call(
        _body,
        grid=(_NT,),
        in_specs=[
            pl.BlockSpec((_T, _D), lambda i: (i, 0)),
            pl.BlockSpec((_T, 1), lambda i: (i, 0)),
            pl.BlockSpec((1, _N), lambda i: (0, 0)),
            pl.BlockSpec((_T, 1), lambda i: (i, 0)),
            pl.BlockSpec((_N, _D), lambda i: (0, 0)),
            pl.BlockSpec((_HI, _LO * _D), lambda i: (0, 0)),
        ],
        out_specs=[
            pl.BlockSpec((_T, _D), lambda i: (i, 0)),
            pl.BlockSpec((1, 1), lambda i: (0, 0)),
            pl.BlockSpec((1, 1), lambda i: (0, 0)),
        ],
        out_shape=[
            jax.ShapeDtypeStruct((_TOKENS, _D), jnp.float32),
            jax.ShapeDtypeStruct((1, 1), jnp.float32),
            jax.ShapeDtypeStruct((1, 1), jnp.float32),
        ],
        scratch_shapes=[
            pltpu.VMEM((_HI, _LO), jnp.float32),
            pltpu.VMEM((1, 1), jnp.float32),
            pltpu.VMEM((1, 1), jnp.float32),
        ],
        compiler_params=pltpu.CompilerParams(
            dimension_semantics=("arbitrary",)),
    )(zf, sz, se, notmask, emb, embt)

    return quant.reshape(input_shape), loss.reshape(()), perp.reshape(())


# speedup vs baseline: 2.9492x; 1.1173x over previous
import functools

import jax
import jax.numpy as jnp
from jax.experimental import pallas as pl
from jax.experimental.pallas import tpu as pltpu

_N = 8192
_D = 32
_TOKENS = 8192
_T = 1024
_NT = _TOKENS // _T
_CHUNK = 2048
_NCHUNK = _N // _CHUNK
_COMMIT = 0.25


_HI = 128
_LO = 64


def _body(zf_ref, sz_ref, se_ref, nm_ref, emb_ref, embt_ref,
          quant_ref, loss_ref, perp_ref,
          counts_ref, lacc_ref, wacc_ref):
    i = pl.program_id(0)

    @pl.when(i == 0)
    def _init():
        counts_ref[...] = jnp.zeros_like(counts_ref)
        lacc_ref[...] = jnp.zeros_like(lacc_ref)
        wacc_ref[...] = jnp.zeros_like(wacc_ref)

    zfb = zf_ref[...]
    zbb = zfb.astype(jnp.bfloat16)
    ebb = emb_ref[...].astype(jnp.bfloat16)
    szb = sz_ref[...]

    acc_v = None
    acc_i = None
    for c in range(_NCHUNK):
        lo = c * _CHUNK
        mm = jax.lax.dot_general(
            zbb, ebb[lo:lo + _CHUNK, :],
            (((1,), (1,)), ((), ())),
            preferred_element_type=jnp.float32)
        dch = (-2.0) * mm + (szb + se_ref[:, lo:lo + _CHUNK])
        mc = jnp.min(dch, axis=1, keepdims=True)
        io = jax.lax.broadcasted_iota(
            jnp.int32, (_T, _CHUNK), 1).astype(jnp.float32)
        icf = jnp.min(jnp.where(dch == mc, io, jnp.float32(_N)),
                      axis=1, keepdims=True)
        ic = icf.astype(jnp.int32) + lo
        if c == 0:
            acc_v = mc.astype(jnp.bfloat16).astype(jnp.float32)
            acc_i = ic
        else:
            keep = acc_v <= mc
            acc_i = jnp.where(keep, acc_i, ic)
            acc_v = jnp.where(keep, acc_v, mc)
            acc_v = acc_v.astype(jnp.bfloat16).astype(jnp.float32)

    hi = jax.lax.shift_right_logical(acc_i, 6)
    lo_ = jax.lax.bitwise_and(acc_i, _LO - 1)
    oh_hi = (jax.lax.broadcasted_iota(jnp.int32, (_T, _HI), 1)
             == hi).astype(jnp.float32)
    oh_lo = (jax.lax.broadcasted_iota(jnp.int32, (_T, _LO), 1)
             == lo_).astype(jnp.bfloat16)

    h = jax.lax.dot_general(
        oh_hi.astype(jnp.bfloat16), embt_ref[...].astype(jnp.bfloat16),
        (((1,), (0,)), ((), ())),
        preferred_element_type=jnp.float32)
    lane = jax.lax.broadcasted_iota(jnp.int32, (_T, _LO * _D), 1)
    masked = jnp.where(jax.lax.shift_right_logical(lane, 5) == lo_,
                       h, 0.0).astype(jnp.bfloat16)
    sel = (jax.lax.bitwise_and(
        jax.lax.broadcasted_iota(jnp.int32, (_LO * _D, _D), 0), _D - 1)
        == jax.lax.broadcasted_iota(jnp.int32, (_LO * _D, _D), 1)
    ).astype(jnp.bfloat16)
    zq = jax.lax.dot_general(
        masked, sel, (((1,), (0,)), ((), ())),
        preferred_element_type=jnp.float32)
    quant_ref[...] = zfb + (zq - zfb)

    lacc_ref[...] += jnp.sum((zq - zfb) ** 2).reshape(1, 1)
    w = nm_ref[...]
    wacc_ref[...] += jnp.sum(w).reshape(1, 1)
    counts_ref[...] += jax.lax.dot_general(
        (oh_hi * w).astype(jnp.bfloat16), oh_lo,
        (((0,), (0,)), ((), ())),
        preferred_element_type=jnp.float32)

    @pl.when(i == _NT - 1)
    def _finish():
        denom = jnp.maximum(wacc_ref[0, 0], 1.0)
        avg = counts_ref[...] / denom
        ent = jnp.sum(avg * jnp.log(avg + 1e-10))
        perp_ref[...] = jnp.exp(-ent).reshape(1, 1)
        loss_ref[...] = (_COMMIT * (lacc_ref[0, 0]
                                    / jnp.float32(_TOKENS * _D))).reshape(1, 1)


@functools.partial(jax.jit, static_argnames=())
def kernel(z, track_pad_mask, emb):
    input_shape = z.shape
    zf = z.reshape(-1, z.shape[-1])
    mask = track_pad_mask.reshape(-1)
    sz = jnp.sum(zf ** 2, axis=1, keepdims=True)
    se = jnp.sum(emb ** 2, axis=1).reshape(1, -1)
    notmask = jnp.logical_not(mask).astype(zf.dtype).reshape(-1, 1)
    embt = emb.reshape(_HI, _LO * _D)

    quant, loss, perp = pl.pallas_call(
        _body,
        grid=(_NT,),
        in_specs=[
            pl.BlockSpec((_T, _D), lambda i: (i, 0)),
            pl.BlockSpec((_T, 1), lambda i: (i, 0)),
            pl.BlockSpec((1, _N), lambda i: (0, 0)),
            pl.BlockSpec((_T, 1), lambda i: (i, 0)),
            pl.BlockSpec((_N, _D), lambda i: (0, 0)),
            pl.BlockSpec((_HI, _LO * _D), lambda i: (0, 0)),
        ],
        out_specs=[
            pl.BlockSpec((_T, _D), lambda i: (i, 0)),
            pl.BlockSpec((1, 1), lambda i: (0, 0)),
            pl.BlockSpec((1, 1), lambda i: (0, 0)),
        ],
        out_shape=[
            jax.ShapeDtypeStruct((_TOKENS, _D), jnp.float32),
            jax.ShapeDtypeStruct((1, 1), jnp.float32),
            jax.ShapeDtypeStruct((1, 1), jnp.float32),
        ],
        scratch_shapes=[
            pltpu.VMEM((_HI, _LO), jnp.float32),
            pltpu.VMEM((1, 1), jnp.float32),
            pltpu.VMEM((1, 1), jnp.float32),
        ],
        compiler_params=pltpu.CompilerParams(
            dimension_semantics=("arbitrary",)),
    )(zf, sz, se, notmask, emb, embt)

    return quant.reshape(input_shape), loss.reshape(()), perp.reshape(())


# speedup vs baseline: 2.9559x; 1.0023x over previous
import functools

import jax
import jax.numpy as jnp
from jax.experimental import pallas as pl
from jax.experimental.pallas import tpu as pltpu

_N = 8192
_D = 32
_TOKENS = 8192
_T = 1024
_NT = _TOKENS // _T
_CHUNK = 2048
_NCHUNK = _N // _CHUNK
_COMMIT = 0.25


_HI = 128
_LO = 64


def _body(zf_ref, sz_ref, se_ref, nm_ref, emb_ref, embt_ref,
          quant_ref, loss_ref, perp_ref,
          counts_ref, lacc_ref, wacc_ref):
    i = pl.program_id(0)

    @pl.when(i == 0)
    def _init():
        counts_ref[...] = jnp.zeros_like(counts_ref)
        lacc_ref[...] = jnp.zeros_like(lacc_ref)
        wacc_ref[...] = jnp.zeros_like(wacc_ref)

    zfb = zf_ref[...]
    zbb = zfb.astype(jnp.bfloat16)
    ebb = emb_ref[...]
    szb = sz_ref[...]

    acc_v = None
    acc_i = None
    for c in range(_NCHUNK):
        lo = c * _CHUNK
        mm = jax.lax.dot_general(
            zbb, ebb[lo:lo + _CHUNK, :],
            (((1,), (1,)), ((), ())),
            preferred_element_type=jnp.float32)
        dch = (-2.0) * mm + (szb + se_ref[:, lo:lo + _CHUNK])
        mc = jnp.min(dch, axis=1, keepdims=True)
        io = jax.lax.broadcasted_iota(
            jnp.int32, (_T, _CHUNK), 1).astype(jnp.float32)
        icf = jnp.min(jnp.where(dch == mc, io, jnp.float32(_N)),
                      axis=1, keepdims=True)
        ic = icf.astype(jnp.int32) + lo
        if c == 0:
            acc_v = mc.astype(jnp.bfloat16).astype(jnp.float32)
            acc_i = ic
        else:
            keep = acc_v <= mc
            acc_i = jnp.where(keep, acc_i, ic)
            acc_v = jnp.where(keep, acc_v, mc)
            acc_v = acc_v.astype(jnp.bfloat16).astype(jnp.float32)

    hi = jax.lax.shift_right_logical(acc_i, 6)
    lo_ = jax.lax.bitwise_and(acc_i, _LO - 1)
    oh_hi = (jax.lax.broadcasted_iota(jnp.int32, (_T, _HI), 1)
             == hi).astype(jnp.float32)
    oh_lo = (jax.lax.broadcasted_iota(jnp.int32, (_T, _LO), 1)
             == lo_).astype(jnp.bfloat16)

    h = jax.lax.dot_general(
        oh_hi.astype(jnp.bfloat16), embt_ref[...],
        (((1,), (0,)), ((), ())),
        preferred_element_type=jnp.float32)
    lane = jax.lax.broadcasted_iota(jnp.int32, (_T, _LO * _D), 1)
    masked = jnp.where(jax.lax.shift_right_logical(lane, 5) == lo_,
                       h, 0.0).astype(jnp.bfloat16)
    sel = (jax.lax.bitwise_and(
        jax.lax.broadcasted_iota(jnp.int32, (_LO * _D, _D), 0), _D - 1)
        == jax.lax.broadcasted_iota(jnp.int32, (_LO * _D, _D), 1)
    ).astype(jnp.bfloat16)
    zq = jax.lax.dot_general(
        masked, sel, (((1,), (0,)), ((), ())),
        preferred_element_type=jnp.float32)
    quant_ref[...] = zfb + (zq - zfb)

    lacc_ref[...] += jnp.sum((zq - zfb) ** 2).reshape(1, 1)
    w = nm_ref[...]
    wacc_ref[...] += jnp.sum(w).reshape(1, 1)
    counts_ref[...] += jax.lax.dot_general(
        (oh_hi * w).astype(jnp.bfloat16), oh_lo,
        (((0,), (0,)), ((), ())),
        preferred_element_type=jnp.float32)

    @pl.when(i == _NT - 1)
    def _finish():
        denom = jnp.maximum(wacc_ref[0, 0], 1.0)
        avg = counts_ref[...] / denom
        ent = jnp.sum(avg * jnp.log(avg + 1e-10))
        perp_ref[...] = jnp.exp(-ent).reshape(1, 1)
        loss_ref[...] = (_COMMIT * (lacc_ref[0, 0]
                                    / jnp.float32(_TOKENS * _D))).reshape(1, 1)


@functools.partial(jax.jit, static_argnames=())
def kernel(z, track_pad_mask, emb):
    input_shape = z.shape
    zf = z.reshape(-1, z.shape[-1])
    mask = track_pad_mask.reshape(-1)
    sz = jnp.sum(zf ** 2, axis=1, keepdims=True)
    se = jnp.sum(emb ** 2, axis=1).reshape(1, -1)
    notmask = jnp.logical_not(mask).astype(zf.dtype).reshape(-1, 1)
    emb_bf = emb.astype(jnp.bfloat16)
    embt = emb_bf.reshape(_HI, _LO * _D)

    quant, loss, perp = pl.pallas_call(
        _body,
        grid=(_NT,),
        in_specs=[
            pl.BlockSpec((_T, _D), lambda i: (i, 0)),
            pl.BlockSpec((_T, 1), lambda i: (i, 0)),
            pl.BlockSpec((1, _N), lambda i: (0, 0)),
            pl.BlockSpec((_T, 1), lambda i: (i, 0)),
            pl.BlockSpec((_N, _D), lambda i: (0, 0)),
            pl.BlockSpec((_HI, _LO * _D), lambda i: (0, 0)),
        ],
        out_specs=[
            pl.BlockSpec((_T, _D), lambda i: (i, 0)),
            pl.BlockSpec((1, 1), lambda i: (0, 0)),
            pl.BlockSpec((1, 1), lambda i: (0, 0)),
        ],
        out_shape=[
            jax.ShapeDtypeStruct((_TOKENS, _D), jnp.float32),
            jax.ShapeDtypeStruct((1, 1), jnp.float32),
            jax.ShapeDtypeStruct((1, 1), jnp.float32),
        ],
        scratch_shapes=[
            pltpu.VMEM((_HI, _LO), jnp.float32),
            pltpu.VMEM((1, 1), jnp.float32),
            pltpu.VMEM((1, 1), jnp.float32),
        ],
        compiler_params=pltpu.CompilerParams(
            dimension_semantics=("arbitrary",)),
    )(zf, sz, se, notmask, emb_bf, embt)

    return quant.reshape(input_shape), loss.reshape(()), perp.reshape(())
